# initial kernel scaffold (unmeasured)
import jax
import jax.numpy as jnp
from jax import lax
from jax.experimental import pallas as pl
from jax.experimental.pallas import tpu as pltpu

N_DEV = 16
AXIS = "i"
CHUNK = 256
REM_BITS = 8


def kernel(x, dest):
    rows, ncol = x.shape
    row_bytes = ncol * jnp.dtype(x.dtype).itemsize
    cnt_msg_bytes = 8 * 128 * 4

    order = jnp.argsort(dest, stable=True)
    x_sorted = jnp.take(x, order, axis=0)
    counts = jnp.bincount(dest, length=N_DEV).astype(jnp.int32)
    cnt_blk = jnp.zeros((8, 128), jnp.int32).at[0, :N_DEV].set(counts)

    def body(x_ref, cnt_ref, out_ref, cnt_mat, cnt_smem, lob, rob,
             cnt_send, cnt_recv, dat_send, dat_recv, loc_sem):
        me = lax.axis_index(AXIS)

        barrier = pltpu.get_barrier_semaphore()
        for p in range(N_DEV):
            @pl.when(me != p)
            def _(p=p):
                pl.semaphore_signal(
                    barrier, inc=1, device_id=(p,),
                    device_id_type=pl.DeviceIdType.MESH)
        pl.semaphore_wait(barrier, N_DEV - 1)

        own = pltpu.make_async_copy(cnt_ref, cnt_mat.at[me], loc_sem)
        own.start()
        own.wait()
        for p in range(N_DEV):
            @pl.when(me != p)
            def _(p=p):
                c = pltpu.make_async_remote_copy(
                    src_ref=cnt_ref, dst_ref=cnt_mat.at[me],
                    send_sem=cnt_send, recv_sem=cnt_recv,
                    device_id=(p,), device_id_type=pl.DeviceIdType.MESH)
                c.start()
        pl.semaphore_wait(cnt_send, (N_DEV - 1) * cnt_msg_bytes)
        pl.semaphore_wait(cnt_recv, (N_DEV - 1) * cnt_msg_bytes)

        bounce = pltpu.make_async_copy(cnt_mat.at[:, 0, :], cnt_smem, loc_sem)
        bounce.start()
        bounce.wait()

        acc = jnp.int32(0)
        for d in range(N_DEV):
            lob[d] = acc
            acc = acc + cnt_smem[me, d]
        for d in range(N_DEV):
            r = jnp.int32(0)
            for s in range(N_DEV):
                r = r + jnp.where(s < me, cnt_smem[s, d], 0)
            rob[d] = r

        def push(lo, ro, c, d, remote):
            nq = c >> 8

            def chunk_body(q, _):
                o = q * CHUNK
                src = x_ref.at[pl.ds(lo + o, CHUNK)]
                dst = out_ref.at[pl.ds(ro + o, CHUNK)]
                if remote:
                    pltpu.make_async_remote_copy(
                        src_ref=src, dst_ref=dst,
                        send_sem=dat_send, recv_sem=dat_recv,
                        device_id=(d,),
                        device_id_type=pl.DeviceIdType.MESH).start()
                else:
                    pltpu.make_async_copy(src, dst, loc_sem).start()
                return 0

            lax.fori_loop(0, nq, chunk_body, 0)
            off = nq * CHUNK
            for b in range(REM_BITS - 1, -1, -1):
                size = 1 << b
                bit = (c >> b) & 1

                @pl.when(bit == 1)
                def _(off=off, size=size):
                    src = x_ref.at[pl.ds(lo + off, size)]
                    dst = out_ref.at[pl.ds(ro + off, size)]
                    if remote:
                        pltpu.make_async_remote_copy(
                            src_ref=src, dst_ref=dst,
                            send_sem=dat_send, recv_sem=dat_recv,
                            device_id=(d,),
                            device_id_type=pl.DeviceIdType.MESH).start()
                    else:
                        pltpu.make_async_copy(src, dst, loc_sem).start()
                off = off + bit * size

        c_self = cnt_smem[me, me]
        push(lob[me], rob[me], c_self, me, remote=False)
        for step in range(1, N_DEV):
            d = lax.rem(me + step, N_DEV)
            push(lob[d], rob[d], cnt_smem[me, d], d, remote=True)

        row_sum = jnp.int32(0)
        col_sum = jnp.int32(0)
        for k in range(N_DEV):
            row_sum = row_sum + cnt_smem[me, k]
            col_sum = col_sum + cnt_smem[k, me]
        pl.semaphore_wait(loc_sem, c_self * row_bytes)
        pl.semaphore_wait(dat_send, (row_sum - c_self) * row_bytes)
        pl.semaphore_wait(dat_recv, (col_sum - c_self) * row_bytes)

    return pl.pallas_call(
        body,
        out_shape=jax.ShapeDtypeStruct((rows, ncol), x.dtype),
        in_specs=[
            pl.BlockSpec(memory_space=pl.ANY),
            pl.BlockSpec(memory_space=pltpu.VMEM),
        ],
        out_specs=pl.BlockSpec(memory_space=pl.ANY),
        scratch_shapes=[
            pltpu.VMEM((N_DEV, 8, 128), jnp.int32),
            pltpu.SMEM((N_DEV, 128), jnp.int32),
            pltpu.SMEM((N_DEV,), jnp.int32),
            pltpu.SMEM((N_DEV,), jnp.int32),
            pltpu.SemaphoreType.DMA,
            pltpu.SemaphoreType.DMA,
            pltpu.SemaphoreType.DMA,
            pltpu.SemaphoreType.DMA,
            pltpu.SemaphoreType.DMA,
        ],
        compiler_params=pltpu.CompilerParams(
            collective_id=0, has_side_effects=True),
    )(x_sorted, cnt_blk)


# baseline (device time: 334643 ns/iter reference)
import jax
import jax.numpy as jnp
from jax import lax
from jax.experimental import pallas as pl
from jax.experimental.pallas import tpu as pltpu

N_DEV = 16
AXIS = "i"
CHUNK = 256
REM_BITS = 8


def kernel(x, dest):
    rows, ncol = x.shape

    assert ncol % 128 == 0 and (ncol // 128) % 8 == 0
    order = jnp.argsort(dest, stable=True)
    x_sorted = jnp.take(x, order, axis=0).reshape(rows, ncol // 128, 128)
    counts = jnp.bincount(dest, length=N_DEV).astype(jnp.int32)
    cnt_blk = jnp.zeros((8, 128), jnp.int32).at[0, :N_DEV].set(counts)

    def body(x_ref, cnt_ref, out_ref, cnt_mat, cnt_smem, lob, rob,
             cnt_send, cnt_recv, dat_send, dat_recv, loc_sem):
        me = lax.axis_index(AXIS)

        barrier = pltpu.get_barrier_semaphore()
        for p in range(N_DEV):
            @pl.when(me != p)
            def _(p=p):
                pl.semaphore_signal(
                    barrier, inc=1, device_id=(p,),
                    device_id_type=pl.DeviceIdType.MESH)
        pl.semaphore_wait(barrier, N_DEV - 1)

        own = pltpu.make_async_copy(cnt_ref, cnt_mat.at[me], loc_sem)
        own.start()
        own.wait()
        for p in range(N_DEV):
            @pl.when(me != p)
            def _(p=p):
                c = pltpu.make_async_remote_copy(
                    src_ref=cnt_ref, dst_ref=cnt_mat.at[me],
                    send_sem=cnt_send, recv_sem=cnt_recv,
                    device_id=(p,), device_id_type=pl.DeviceIdType.MESH)
                c.start()
        for _ in range(N_DEV - 1):
            m = pltpu.make_async_remote_copy(
                src_ref=cnt_ref, dst_ref=cnt_mat.at[0],
                send_sem=cnt_send, recv_sem=cnt_recv,
                device_id=(0,), device_id_type=pl.DeviceIdType.MESH)
            m.wait_send()
            m.wait_recv()

        bounce = pltpu.make_async_copy(cnt_mat.at[:, 0, :], cnt_smem, loc_sem)
        bounce.start()
        bounce.wait()

        acc = jnp.int32(0)
        for d in range(N_DEV):
            lob[d] = acc
            acc = acc + cnt_smem[me, d]
        for d in range(N_DEV):
            r = jnp.int32(0)
            for s in range(N_DEV):
                r = r + jnp.where(s < me, cnt_smem[s, d], 0)
            rob[d] = r

        def push(lo, ro, c, d, remote):
            nq = c >> 8

            def chunk_body(q, _):
                o = q * CHUNK
                src = x_ref.at[pl.ds(lo + o, CHUNK)]
                dst = out_ref.at[pl.ds(ro + o, CHUNK)]
                if remote:
                    pltpu.make_async_remote_copy(
                        src_ref=src, dst_ref=dst,
                        send_sem=dat_send, recv_sem=dat_recv,
                        device_id=(d,),
                        device_id_type=pl.DeviceIdType.MESH).start()
                else:
                    cp = pltpu.make_async_copy(src, dst, loc_sem)
                    cp.start()
                    cp.wait()
                return 0

            lax.fori_loop(0, nq, chunk_body, 0)
            off = nq * CHUNK
            for b in range(REM_BITS - 1, -1, -1):
                size = 1 << b
                bit = (c >> b) & 1

                @pl.when(bit == 1)
                def _(off=off, size=size):
                    src = x_ref.at[pl.ds(lo + off, size)]
                    dst = out_ref.at[pl.ds(ro + off, size)]
                    if remote:
                        pltpu.make_async_remote_copy(
                            src_ref=src, dst_ref=dst,
                            send_sem=dat_send, recv_sem=dat_recv,
                            device_id=(d,),
                            device_id_type=pl.DeviceIdType.MESH).start()
                    else:
                        cp = pltpu.make_async_copy(src, dst, loc_sem)
                        cp.start()
                        cp.wait()
                off = off + bit * size

        def drain(c, is_send):
            def mirror(size):
                m = pltpu.make_async_remote_copy(
                    src_ref=x_ref.at[pl.ds(0, size)],
                    dst_ref=out_ref.at[pl.ds(0, size)],
                    send_sem=dat_send, recv_sem=dat_recv,
                    device_id=(0,), device_id_type=pl.DeviceIdType.MESH)
                if is_send:
                    m.wait_send()
                else:
                    m.wait_recv()

            def chunk_body(q, _):
                mirror(CHUNK)
                return 0

            lax.fori_loop(0, c >> 8, chunk_body, 0)
            for b in range(REM_BITS - 1, -1, -1):
                @pl.when(((c >> b) & 1) == 1)
                def _(b=b):
                    mirror(1 << b)

        c_self = cnt_smem[me, me]
        push(lob[me], rob[me], c_self, me, remote=False)
        for step in range(1, N_DEV):
            d = lax.rem(me + step, N_DEV)
            push(lob[d], rob[d], cnt_smem[me, d], d, remote=True)

        for step in range(1, N_DEV):
            d = lax.rem(me + step, N_DEV)
            drain(cnt_smem[me, d], is_send=True)
        for s in range(N_DEV):
            c_in = jnp.where(s == me, 0, cnt_smem[s, me])
            drain(c_in, is_send=False)

    out = pl.pallas_call(
        body,
        out_shape=jax.ShapeDtypeStruct((rows, ncol // 128, 128), x.dtype),
        in_specs=[
            pl.BlockSpec(memory_space=pltpu.HBM),
            pl.BlockSpec(memory_space=pltpu.VMEM),
        ],
        out_specs=pl.BlockSpec(memory_space=pltpu.HBM),
        scratch_shapes=[
            pltpu.VMEM((N_DEV, 8, 128), jnp.int32),
            pltpu.SMEM((N_DEV, 128), jnp.int32),
            pltpu.SMEM((N_DEV,), jnp.int32),
            pltpu.SMEM((N_DEV,), jnp.int32),
            pltpu.SemaphoreType.DMA,
            pltpu.SemaphoreType.DMA,
            pltpu.SemaphoreType.DMA,
            pltpu.SemaphoreType.DMA,
            pltpu.SemaphoreType.DMA,
        ],
        compiler_params=pltpu.CompilerParams(
            collective_id=0, has_side_effects=True),
    )(x_sorted, cnt_blk)
    return out.reshape(rows, ncol)


# device time: 284529 ns/iter; 1.1761x vs baseline; 1.1761x over previous
import jax
import jax.numpy as jnp
from jax import lax
from jax.experimental import pallas as pl
from jax.experimental.pallas import tpu as pltpu

N_DEV = 16
AXIS = "i"
CHUNK = 256
REM_BITS = 8


def kernel(x, dest):
    rows, ncol = x.shape
    assert ncol % 128 == 0 and (ncol // 128) % 8 == 0
    x3 = x.reshape(rows, ncol // 128, 128)
    dest = dest.astype(jnp.int32)

    def body(x_ref, dest_ref, out_ref, xs_ref, cnt_mat, cnt_smem, slot_ref,
             counters, lob, rob, cnt_send, cnt_recv, dat_send, dat_recv,
             loc_sem):
        me = lax.axis_index(AXIS)

        for k in range(N_DEV):
            counters[k] = jnp.int32(0)

        def scan_body(j, _):
            d = dest_ref[j]
            c = counters[d]
            slot_ref[j] = c
            counters[d] = c + 1
            return 0

        lax.fori_loop(0, rows, scan_body, 0)

        barrier = pltpu.get_barrier_semaphore()
        for p in range(N_DEV):
            @pl.when(me != p)
            def _(p=p):
                pl.semaphore_signal(
                    barrier, inc=1, device_id=(p,),
                    device_id_type=pl.DeviceIdType.MESH)
        pl.semaphore_wait(barrier, N_DEV - 1)

        own = pltpu.make_async_copy(counters, cnt_mat.at[me], loc_sem)
        own.start()
        own.wait()
        for p in range(N_DEV):
            @pl.when(me != p)
            def _(p=p):
                c = pltpu.make_async_remote_copy(
                    src_ref=cnt_mat.at[me], dst_ref=cnt_mat.at[me],
                    send_sem=cnt_send, recv_sem=cnt_recv,
                    device_id=(p,), device_id_type=pl.DeviceIdType.MESH)
                c.start()
        for _ in range(N_DEV - 1):
            m = pltpu.make_async_remote_copy(
                src_ref=cnt_mat.at[0], dst_ref=cnt_mat.at[0],
                send_sem=cnt_send, recv_sem=cnt_recv,
                device_id=(0,), device_id_type=pl.DeviceIdType.MESH)
            m.wait_send()
            m.wait_recv()

        bounce = pltpu.make_async_copy(cnt_mat, cnt_smem, loc_sem)
        bounce.start()
        bounce.wait()

        acc = jnp.int32(0)
        for d in range(N_DEV):
            lob[d] = acc
            acc = acc + cnt_smem[me, d]
        for d in range(N_DEV):
            r = jnp.int32(0)
            for s in range(N_DEV):
                r = r + jnp.where(s < me, cnt_smem[s, d], 0)
            rob[d] = r

        def gather_body(j, _):
            d = dest_ref[j]
            xs_ref[lob[d] + slot_ref[j]] = x_ref[j]
            return 0

        lax.fori_loop(0, rows, gather_body, 0)

        def push(lo, ro, c, d, remote):
            nq = c >> 8

            def chunk_body(q, _):
                o = q * CHUNK
                src = xs_ref.at[pl.ds(lo + o, CHUNK)]
                dst = out_ref.at[pl.ds(ro + o, CHUNK)]
                if remote:
                    pltpu.make_async_remote_copy(
                        src_ref=src, dst_ref=dst,
                        send_sem=dat_send, recv_sem=dat_recv,
                        device_id=(d,),
                        device_id_type=pl.DeviceIdType.MESH).start()
                else:
                    cp = pltpu.make_async_copy(src, dst, loc_sem)
                    cp.start()
                    cp.wait()
                return 0

            lax.fori_loop(0, nq, chunk_body, 0)
            off = nq * CHUNK
            for b in range(REM_BITS - 1, -1, -1):
                size = 1 << b
                bit = (c >> b) & 1

                @pl.when(bit == 1)
                def _(off=off, size=size):
                    src = xs_ref.at[pl.ds(lo + off, size)]
                    dst = out_ref.at[pl.ds(ro + off, size)]
                    if remote:
                        pltpu.make_async_remote_copy(
                            src_ref=src, dst_ref=dst,
                            send_sem=dat_send, recv_sem=dat_recv,
                            device_id=(d,),
                            device_id_type=pl.DeviceIdType.MESH).start()
                    else:
                        cp = pltpu.make_async_copy(src, dst, loc_sem)
                        cp.start()
                        cp.wait()
                off = off + bit * size

        def drain(c, is_send):
            def mirror(size):
                m = pltpu.make_async_remote_copy(
                    src_ref=xs_ref.at[pl.ds(0, size)],
                    dst_ref=out_ref.at[pl.ds(0, size)],
                    send_sem=dat_send, recv_sem=dat_recv,
                    device_id=(0,), device_id_type=pl.DeviceIdType.MESH)
                if is_send:
                    m.wait_send()
                else:
                    m.wait_recv()

            def chunk_body(q, _):
                mirror(CHUNK)
                return 0

            lax.fori_loop(0, c >> 8, chunk_body, 0)
            for b in range(REM_BITS - 1, -1, -1):
                @pl.when(((c >> b) & 1) == 1)
                def _(b=b):
                    mirror(1 << b)

        push(lob[me], rob[me], cnt_smem[me, me], me, remote=False)
        for step in range(1, N_DEV):
            d = lax.rem(me + step, N_DEV)
            push(lob[d], rob[d], cnt_smem[me, d], d, remote=True)

        for step in range(1, N_DEV):
            d = lax.rem(me + step, N_DEV)
            drain(cnt_smem[me, d], is_send=True)
        for s in range(N_DEV):
            c_in = jnp.where(s == me, 0, cnt_smem[s, me])
            drain(c_in, is_send=False)

    out = pl.pallas_call(
        body,
        out_shape=jax.ShapeDtypeStruct((rows, ncol // 128, 128), x.dtype),
        in_specs=[
            pl.BlockSpec(memory_space=pltpu.VMEM),
            pl.BlockSpec(memory_space=pltpu.SMEM),
        ],
        out_specs=pl.BlockSpec(memory_space=pltpu.HBM),
        scratch_shapes=[
            pltpu.VMEM((rows, ncol // 128, 128), x.dtype),
            pltpu.VMEM((N_DEV, 128), jnp.int32),
            pltpu.SMEM((N_DEV, 128), jnp.int32),
            pltpu.SMEM((rows,), jnp.int32),
            pltpu.SMEM((128,), jnp.int32),
            pltpu.SMEM((N_DEV,), jnp.int32),
            pltpu.SMEM((N_DEV,), jnp.int32),
            pltpu.SemaphoreType.DMA,
            pltpu.SemaphoreType.DMA,
            pltpu.SemaphoreType.DMA,
            pltpu.SemaphoreType.DMA,
            pltpu.SemaphoreType.DMA,
        ],
        compiler_params=pltpu.CompilerParams(
            collective_id=0, has_side_effects=True,
            vmem_limit_bytes=48 * 1024 * 1024),
    )(x3, dest)
    return out.reshape(rows, ncol)


# device time: 277066 ns/iter; 1.2078x vs baseline; 1.0269x over previous
import jax
import jax.numpy as jnp
from jax import lax
from jax.experimental import pallas as pl
from jax.experimental.pallas import tpu as pltpu

N_DEV = 16
AXIS = "i"
CHUNK = 256
REM_BITS = 8


def kernel(x, dest):
    rows, ncol = x.shape
    assert ncol % 128 == 0 and (ncol // 128) % 8 == 0
    x3 = x.reshape(rows, ncol // 128, 128)
    dest = dest.astype(jnp.int32)

    def body(x_ref, dest_ref, out_ref, xs_ref, cnt_mat, cnt_smem, slot_ref,
             counters, lob, rob, cnt_send, cnt_recv, dat_send, dat_recv,
             loc_sem):
        me = lax.axis_index(AXIS)

        for k in range(N_DEV):
            counters[k] = jnp.int32(0)

        def scan_body(j, _):
            d = dest_ref[j]
            c = counters[d]
            slot_ref[j] = c
            counters[d] = c + 1
            return 0

        lax.fori_loop(0, rows, scan_body, 0)

        acc = jnp.int32(0)
        for d in range(N_DEV):
            lob[d] = acc
            acc = acc + counters[d]

        barrier = pltpu.get_barrier_semaphore()
        for p in range(N_DEV):
            @pl.when(me != p)
            def _(p=p):
                pl.semaphore_signal(
                    barrier, inc=1, device_id=(p,),
                    device_id_type=pl.DeviceIdType.MESH)
        pl.semaphore_wait(barrier, N_DEV - 1)

        own = pltpu.make_async_copy(counters, cnt_mat.at[me], loc_sem)
        own.start()
        own.wait()
        for p in range(N_DEV):
            @pl.when(me != p)
            def _(p=p):
                c = pltpu.make_async_remote_copy(
                    src_ref=cnt_mat.at[me], dst_ref=cnt_mat.at[me],
                    send_sem=cnt_send, recv_sem=cnt_recv,
                    device_id=(p,), device_id_type=pl.DeviceIdType.MESH)
                c.start()
        def gather_body(j, _):
            d = dest_ref[j]
            xs_ref[lob[d] + slot_ref[j]] = x_ref[j]
            return 0

        lax.fori_loop(0, rows, gather_body, 0)

        for _ in range(N_DEV - 1):
            m = pltpu.make_async_remote_copy(
                src_ref=cnt_mat.at[0], dst_ref=cnt_mat.at[0],
                send_sem=cnt_send, recv_sem=cnt_recv,
                device_id=(0,), device_id_type=pl.DeviceIdType.MESH)
            m.wait_send()
            m.wait_recv()

        bounce = pltpu.make_async_copy(cnt_mat, cnt_smem, loc_sem)
        bounce.start()
        bounce.wait()

        for d in range(N_DEV):
            r = jnp.int32(0)
            for s in range(N_DEV):
                r = r + jnp.where(s < me, cnt_smem[s, d], 0)
            rob[d] = r

        def push(lo, ro, c, d, remote):
            nq = c >> 8

            def chunk_body(q, _):
                o = q * CHUNK
                src = xs_ref.at[pl.ds(lo + o, CHUNK)]
                dst = out_ref.at[pl.ds(ro + o, CHUNK)]
                if remote:
                    pltpu.make_async_remote_copy(
                        src_ref=src, dst_ref=dst,
                        send_sem=dat_send, recv_sem=dat_recv,
                        device_id=(d,),
                        device_id_type=pl.DeviceIdType.MESH).start()
                else:
                    cp = pltpu.make_async_copy(src, dst, loc_sem)
                    cp.start()
                    cp.wait()
                return 0

            lax.fori_loop(0, nq, chunk_body, 0)
            off = nq * CHUNK
            for b in range(REM_BITS - 1, -1, -1):
                size = 1 << b
                bit = (c >> b) & 1

                @pl.when(bit == 1)
                def _(off=off, size=size):
                    src = xs_ref.at[pl.ds(lo + off, size)]
                    dst = out_ref.at[pl.ds(ro + off, size)]
                    if remote:
                        pltpu.make_async_remote_copy(
                            src_ref=src, dst_ref=dst,
                            send_sem=dat_send, recv_sem=dat_recv,
                            device_id=(d,),
                            device_id_type=pl.DeviceIdType.MESH).start()
                    else:
                        cp = pltpu.make_async_copy(src, dst, loc_sem)
                        cp.start()
                        cp.wait()
                off = off + bit * size

        def drain(c, is_send):
            def mirror(size):
                m = pltpu.make_async_remote_copy(
                    src_ref=xs_ref.at[pl.ds(0, size)],
                    dst_ref=out_ref.at[pl.ds(0, size)],
                    send_sem=dat_send, recv_sem=dat_recv,
                    device_id=(0,), device_id_type=pl.DeviceIdType.MESH)
                if is_send:
                    m.wait_send()
                else:
                    m.wait_recv()

            def chunk_body(q, _):
                mirror(CHUNK)
                return 0

            lax.fori_loop(0, c >> 8, chunk_body, 0)
            for b in range(REM_BITS - 1, -1, -1):
                @pl.when(((c >> b) & 1) == 1)
                def _(b=b):
                    mirror(1 << b)

        for step in range(1, N_DEV):
            d = lax.rem(me + step, N_DEV)
            push(lob[d], rob[d], cnt_smem[me, d], d, remote=True)
        push(lob[me], rob[me], cnt_smem[me, me], me, remote=False)

        for step in range(1, N_DEV):
            d = lax.rem(me + step, N_DEV)
            drain(cnt_smem[me, d], is_send=True)
        for s in range(N_DEV):
            c_in = jnp.where(s == me, 0, cnt_smem[s, me])
            drain(c_in, is_send=False)

    out = pl.pallas_call(
        body,
        out_shape=jax.ShapeDtypeStruct((rows, ncol // 128, 128), x.dtype),
        in_specs=[
            pl.BlockSpec(memory_space=pltpu.VMEM),
            pl.BlockSpec(memory_space=pltpu.SMEM),
        ],
        out_specs=pl.BlockSpec(memory_space=pltpu.HBM),
        scratch_shapes=[
            pltpu.VMEM((rows, ncol // 128, 128), x.dtype),
            pltpu.VMEM((N_DEV, 128), jnp.int32),
            pltpu.SMEM((N_DEV, 128), jnp.int32),
            pltpu.SMEM((rows,), jnp.int32),
            pltpu.SMEM((128,), jnp.int32),
            pltpu.SMEM((N_DEV,), jnp.int32),
            pltpu.SMEM((N_DEV,), jnp.int32),
            pltpu.SemaphoreType.DMA,
            pltpu.SemaphoreType.DMA,
            pltpu.SemaphoreType.DMA,
            pltpu.SemaphoreType.DMA,
            pltpu.SemaphoreType.DMA,
        ],
        compiler_params=pltpu.CompilerParams(
            collective_id=0, has_side_effects=True,
            vmem_limit_bytes=48 * 1024 * 1024),
    )(x3, dest)
    return out.reshape(rows, ncol)
